# baseline (speedup 1.0000x reference)
import functools
import math

import jax
import jax.numpy as jnp
from jax import lax
from jax.experimental import pallas as pl
from jax.experimental.pallas import tpu as pltpu
from jax.experimental.pallas import tpu_sc as plsc

VOCAB = 1000000
SEQ = 2048
DIM = 64
B = 4

NC = 2
NS = 16
LANES = 16
NW = NC * NS
TOK_PER_W = B * SEQ // NW
CHUNKS_PER_SEQ = SEQ // TOK_PER_W
TCOL = 128
NBUF = 8
N_STEPS = TOK_PER_W // NBUF

_SCALE = math.sqrt(DIM)


def _emb_body(idx_hbm, tokT_hbm, posT_hbm, outT_hbm, idx_v, col_buf, gath_v,
              pos_v, gsems, psem):
    wid = lax.axis_index("s") * NC + lax.axis_index("c")
    b = wid // CHUNKS_PER_SEQ
    s0 = lax.rem(wid, CHUNKS_PER_SEQ) * TOK_PER_W

    pltpu.sync_copy(idx_hbm.at[b, pl.ds(s0, TOK_PER_W)],
                    idx_v.at[pl.ds(0, TOK_PER_W)])

    pcopy = pltpu.async_copy(posT_hbm.at[:, pl.ds(s0, TOK_PER_W)], pos_v,
                             psem)

    lanes_i = lax.iota(jnp.int32, LANES)

    def _col_off(r):
        return pl.multiple_of((r >> 7) << 7, TCOL)

    def _fire(slot, r):
        pltpu.async_copy(tokT_hbm.at[:, pl.ds(_col_off(r), TCOL)],
                         col_buf.at[slot], gsems.at[slot])

    idx_vec0 = idx_v[pl.ds(0, LANES)]
    for j in range(NBUF):
        _fire(j, idx_vec0[j])

    def _consume(i, j, idx_vec):
        t = i * NBUF + j
        r = idx_vec[j]
        pltpu.make_async_copy(tokT_hbm.at[:, pl.ds(0, TCOL)],
                              col_buf.at[j], gsems.at[j]).wait()
        rm_v = jnp.full((LANES,), r & (TCOL - 1), dtype=jnp.int32)
        t_v = jnp.full((LANES,), t, dtype=jnp.int32)
        for g in range(DIM // LANES):
            ch = lanes_i + g * LANES
            vals = plsc.load_gather(col_buf.at[j], [ch, rm_v])
            plsc.store_scatter(gath_v, [ch, t_v], vals)

    def step(i, carry):
        idx_vec = idx_v[pl.ds(i * NBUF, 2 * NBUF)]
        for j in range(NBUF):
            _consume(i, j, idx_vec)
            _fire(j, idx_vec[NBUF + j])
        return carry

    lax.fori_loop(0, N_STEPS - 1, step, 0)
    idx_vec_last = idx_v[pl.ds((N_STEPS - 1) * NBUF, 2 * NBUF)]
    for j in range(NBUF):
        _consume(N_STEPS - 1, j, idx_vec_last)

    pcopy.wait()

    def row_step(c, carry):
        for j in range(TOK_PER_W // LANES):
            sl = pl.ds(j * LANES, LANES)
            gath_v[c, sl] = (gath_v[c, sl] + pos_v[c, sl]) * _SCALE
        return carry

    lax.fori_loop(0, DIM, row_step, 0, unroll=2)

    pltpu.sync_copy(gath_v, outT_hbm.at[b, :, pl.ds(s0, TOK_PER_W)])


@functools.partial(jax.jit, static_argnames=())
def kernel(x_bs, tok_weight, pos_weight):
    tokT = tok_weight.T
    posT = pos_weight.T
    mesh = plsc.VectorSubcoreMesh(core_axis_name="c", subcore_axis_name="s",
                                  num_cores=NC, num_subcores=NS)
    outT = pl.kernel(
        _emb_body,
        out_type=jax.ShapeDtypeStruct((B, DIM, SEQ), jnp.float32),
        mesh=mesh,
        scratch_types=[
            pltpu.VMEM((TOK_PER_W + 2 * NBUF,), jnp.int32),
            pltpu.VMEM((NBUF, DIM, TCOL), jnp.float32),
            pltpu.VMEM((DIM, TOK_PER_W), jnp.float32),
            pltpu.VMEM((DIM, TOK_PER_W), jnp.float32),
            pltpu.SemaphoreType.DMA((NBUF,)),
            pltpu.SemaphoreType.DMA,
        ],
        compiler_params=pltpu.CompilerParams(needs_layout_passes=False),
        name="input_embedding_sc",
    )(x_bs, tokT, posT)
    return outT.transpose(0, 2, 1)


# speedup vs baseline: 1.0038x; 1.0038x over previous
import functools
import math

import jax
import jax.numpy as jnp
from jax import lax
from jax.experimental import pallas as pl
from jax.experimental.pallas import tpu as pltpu
from jax.experimental.pallas import tpu_sc as plsc

VOCAB = 1000000
SEQ = 2048
DIM = 64
B = 4

NC = 2
NS = 16
LANES = 16
NW = NC * NS
TOK_PER_W = B * SEQ // NW
CHUNKS_PER_SEQ = SEQ // TOK_PER_W
TCOL = 128
NBUF = 8
N_STEPS = TOK_PER_W // NBUF

_SCALE = math.sqrt(DIM)


def _emb_body(idx_hbm, tokT_hbm, posT_hbm, outT_hbm, idx_v, col_buf, gath_v,
              pos_v, gsems, psem):
    wid = lax.axis_index("s") * NC + lax.axis_index("c")
    b = wid // CHUNKS_PER_SEQ
    s0 = lax.rem(wid, CHUNKS_PER_SEQ) * TOK_PER_W

    pltpu.sync_copy(idx_hbm.at[b, pl.ds(s0, TOK_PER_W)],
                    idx_v.at[pl.ds(0, TOK_PER_W)])

    pcopy = pltpu.async_copy(posT_hbm.at[:, pl.ds(s0, TOK_PER_W)], pos_v,
                             psem)

    lanes_i = lax.iota(jnp.int32, LANES)

    def _col_off(r):
        return pl.multiple_of((r >> 7) << 7, TCOL)

    def _fire(slot, r):
        off = _col_off(r)
        for tr in range(DIM // 8):
            pltpu.async_copy(
                tokT_hbm.at[pl.ds(tr * 8, 8), pl.ds(off, TCOL)],
                col_buf.at[slot, pl.ds(tr * 8, 8), :],
                gsems.at[slot])

    idx_vec0 = idx_v[pl.ds(0, LANES)]
    for j in range(NBUF):
        _fire(j, idx_vec0[j])

    def _consume(i, j, idx_vec):
        t = i * NBUF + j
        r = idx_vec[j]
        pltpu.make_async_copy(tokT_hbm.at[:, pl.ds(0, TCOL)],
                              col_buf.at[j], gsems.at[j]).wait()
        rm_v = jnp.full((LANES,), r & (TCOL - 1), dtype=jnp.int32)
        t_v = jnp.full((LANES,), t, dtype=jnp.int32)
        for g in range(DIM // LANES):
            ch = lanes_i + g * LANES
            vals = plsc.load_gather(col_buf.at[j], [ch, rm_v])
            plsc.store_scatter(gath_v, [ch, t_v], vals)

    def step(i, carry):
        idx_vec = idx_v[pl.ds(i * NBUF, 2 * NBUF)]
        for j in range(NBUF):
            _consume(i, j, idx_vec)
            _fire(j, idx_vec[NBUF + j])
        return carry

    lax.fori_loop(0, N_STEPS - 1, step, 0)
    idx_vec_last = idx_v[pl.ds((N_STEPS - 1) * NBUF, 2 * NBUF)]
    for j in range(NBUF):
        _consume(N_STEPS - 1, j, idx_vec_last)

    pcopy.wait()

    def row_step(c, carry):
        for j in range(TOK_PER_W // LANES):
            sl = pl.ds(j * LANES, LANES)
            gath_v[c, sl] = (gath_v[c, sl] + pos_v[c, sl]) * _SCALE
        return carry

    lax.fori_loop(0, DIM, row_step, 0, unroll=2)

    pltpu.sync_copy(gath_v, outT_hbm.at[b, :, pl.ds(s0, TOK_PER_W)])


@functools.partial(jax.jit, static_argnames=())
def kernel(x_bs, tok_weight, pos_weight):
    tokT = tok_weight.T
    posT = pos_weight.T
    mesh = plsc.VectorSubcoreMesh(core_axis_name="c", subcore_axis_name="s",
                                  num_cores=NC, num_subcores=NS)
    outT = pl.kernel(
        _emb_body,
        out_type=jax.ShapeDtypeStruct((B, DIM, SEQ), jnp.float32),
        mesh=mesh,
        scratch_types=[
            pltpu.VMEM((TOK_PER_W + 2 * NBUF,), jnp.int32),
            pltpu.VMEM((NBUF, DIM, TCOL), jnp.float32),
            pltpu.VMEM((DIM, TOK_PER_W), jnp.float32),
            pltpu.VMEM((DIM, TOK_PER_W), jnp.float32),
            pltpu.SemaphoreType.DMA((NBUF,)),
            pltpu.SemaphoreType.DMA,
        ],
        compiler_params=pltpu.CompilerParams(needs_layout_passes=False),
        name="input_embedding_sc",
    )(x_bs, tokT, posT)
    return outT.transpose(0, 2, 1)


# speedup vs baseline: 1.0106x; 1.0067x over previous
import functools
import math

import jax
import jax.numpy as jnp
from jax import lax
from jax.experimental import pallas as pl
from jax.experimental.pallas import tpu as pltpu
from jax.experimental.pallas import tpu_sc as plsc

VOCAB = 1000000
SEQ = 2048
DIM = 64
B = 4

NC = 2
NS = 16
LANES = 16
NW = NC * NS
TOK_PER_W = B * SEQ // NW
CHUNKS_PER_SEQ = SEQ // TOK_PER_W
TCOL = 128
NBUF = 8
N_STEPS = TOK_PER_W // NBUF

_SCALE = math.sqrt(DIM)


def _emb_body(idx_hbm, tokT_hbm, posT_hbm, outT_hbm, idx_v, col_buf, gath_v,
              pos_v, gsems, psem):
    wid = lax.axis_index("s") * NC + lax.axis_index("c")
    b = wid // CHUNKS_PER_SEQ
    s0 = lax.rem(wid, CHUNKS_PER_SEQ) * TOK_PER_W

    pltpu.sync_copy(idx_hbm.at[b, pl.ds(s0, TOK_PER_W)],
                    idx_v.at[pl.ds(0, TOK_PER_W)])

    pcopy = pltpu.async_copy(posT_hbm.at[:, pl.ds(s0, TOK_PER_W)], pos_v,
                             psem)

    lanes_i = lax.iota(jnp.int32, LANES)

    def _col_off(r):
        return pl.multiple_of((r >> 7) << 7, TCOL)

    def _fire(slot, r):
        pltpu.async_copy(tokT_hbm.at[:, pl.ds(_col_off(r), TCOL)],
                         col_buf.at[slot], gsems.at[slot])

    idx_vec0 = idx_v[pl.ds(0, LANES)]
    for j in range(NBUF):
        _fire(j, idx_vec0[j])
    pcopy.wait()

    def _consume(i, j, idx_vec):
        t = i * NBUF + j
        r = idx_vec[j]
        pltpu.make_async_copy(tokT_hbm.at[:, pl.ds(0, TCOL)],
                              col_buf.at[j], gsems.at[j]).wait()
        rm_v = jnp.full((LANES,), r & (TCOL - 1), dtype=jnp.int32)
        t_v = jnp.full((LANES,), t, dtype=jnp.int32)
        for g in range(DIM // LANES):
            ch = lanes_i + g * LANES
            vals = plsc.load_gather(col_buf.at[j], [ch, rm_v])
            pvals = plsc.load_gather(pos_v, [ch, t_v])
            plsc.store_scatter(gath_v, [ch, t_v], (vals + pvals) * _SCALE)

    def step(i, carry):
        idx_vec = idx_v[pl.ds(i * NBUF, 2 * NBUF)]
        for j in range(NBUF):
            _consume(i, j, idx_vec)
            _fire(j, idx_vec[NBUF + j])
        return carry

    lax.fori_loop(0, N_STEPS - 1, step, 0)
    idx_vec_last = idx_v[pl.ds((N_STEPS - 1) * NBUF, 2 * NBUF)]
    for j in range(NBUF):
        _consume(N_STEPS - 1, j, idx_vec_last)

    pltpu.sync_copy(gath_v, outT_hbm.at[b, :, pl.ds(s0, TOK_PER_W)])


@functools.partial(jax.jit, static_argnames=())
def kernel(x_bs, tok_weight, pos_weight):
    tokT = tok_weight.T
    posT = pos_weight.T
    mesh = plsc.VectorSubcoreMesh(core_axis_name="c", subcore_axis_name="s",
                                  num_cores=NC, num_subcores=NS)
    outT = pl.kernel(
        _emb_body,
        out_type=jax.ShapeDtypeStruct((B, DIM, SEQ), jnp.float32),
        mesh=mesh,
        scratch_types=[
            pltpu.VMEM((TOK_PER_W + 2 * NBUF,), jnp.int32),
            pltpu.VMEM((NBUF, DIM, TCOL), jnp.float32),
            pltpu.VMEM((DIM, TOK_PER_W), jnp.float32),
            pltpu.VMEM((DIM, TOK_PER_W), jnp.float32),
            pltpu.SemaphoreType.DMA((NBUF,)),
            pltpu.SemaphoreType.DMA,
        ],
        compiler_params=pltpu.CompilerParams(needs_layout_passes=False),
        name="input_embedding_sc",
    )(x_bs, tokT, posT)
    return outT.transpose(0, 2, 1)
